# baseline (speedup 1.0000x reference)
import functools
import jax
import jax.numpy as jnp
from jax import lax
from jax.experimental import pallas as pl
from jax.experimental.pallas import tpu as pltpu
from jax.experimental.pallas import tpu_sc as plsc

_F32 = jnp.float32
_I32 = jnp.int32
_H = 8
_NC = 2
_NS = 16
_NW = _NC * _NS
_W128 = 128


def _erf(x):
    p = 0.3275911
    a1, a2, a3, a4, a5 = (0.254829592, -0.284496736, 1.421413741,
                          -1.453152027, 1.061405429)
    ax = jnp.abs(x)
    t = 1.0 / (1.0 + p * ax)
    poly = ((((a5 * t + a4) * t + a3) * t + a2) * t + a1) * t
    y = 1.0 - poly * jnp.exp(-ax * ax)
    return jnp.sign(x) * y


def _gelu(x):
    return 0.5 * x * (1.0 + _erf(x * 0.7071067811865475))


def _ln(x, g, b, eps=1e-3):
    mu = jnp.mean(x, axis=-1, keepdims=True)
    var = jnp.mean((x - mu) ** 2, axis=-1, keepdims=True)
    return (x - mu) / jnp.sqrt(var + eps) * g + b


def _prep_body(D, nodes_ref, w_ref, t_ref):
    n = nodes_ref[0]
    t_ref[0, :, :D] = jnp.dot(n, w_ref[0:D], preferred_element_type=_F32)
    t_ref[0, :, D:2 * D] = jnp.dot(n, w_ref[D:2 * D],
                                   preferred_element_type=_F32)


def _sc_gather_body(B, N, E, F, CH, t_hbm, e_hbm, g_hbm,
                    e2v, idxv, rows, gbuf, sem):
    epw = (B * E) // _NW
    nch = epw // CH
    wid = lax.axis_index("s") * _NC + lax.axis_index("c")
    base_e = wid * epw
    nbase = (base_e // E) * N

    def chunk(k, carry):
        eoff = base_e + k * CH
        pltpu.sync_copy(e_hbm.at[pl.ds(eoff * 2, 2 * CH)], e2v)

        def mkidx(i, c):
            v = e2v[pl.ds(i * 16, 16)]
            idxv[i // 8, pl.ds((i % 8) * 16, 16)] = v + nbase
            return c

        lax.fori_loop(0, (2 * CH) // 16, mkidx, 0, unroll=4)
        cps = [pltpu.async_copy(t_hbm.at[idxv.at[j]],
                                rows.at[pl.ds(j * 128, 128)], sem)
               for j in range((2 * CH) // 128)]
        for cp in cps:
            cp.wait()

        def pairadd(e, c):
            for u in range(F // 16):
                sl = pl.ds(u * 16, 16)
                gbuf[e, sl] = rows[2 * e, sl] + rows[2 * e + 1,
                                                     pl.ds(F + u * 16, 16)]
            return c

        lax.fori_loop(0, CH, pairadd, 0, unroll=2)
        pltpu.sync_copy(gbuf, g_hbm.at[pl.ds(eoff, CH)])
        return carry

    lax.fori_loop(0, nch, chunk, 0)


def _sc_gather(B, N, E, F, t2, eflat):
    CH = 256
    kfn = functools.partial(_sc_gather_body, B, N, E, F, CH)
    return pl.kernel(
        kfn,
        mesh=plsc.VectorSubcoreMesh(core_axis_name="c", subcore_axis_name="s"),
        out_type=jax.ShapeDtypeStruct((B * E, F), _F32),
        scratch_types=[
            pltpu.VMEM((2 * CH,), _I32),
            pltpu.VMEM(((2 * CH) // 128, 128), _I32),
            pltpu.VMEM((2 * CH, _W128), _F32),
            pltpu.VMEM((CH, F), _F32),
            pltpu.SemaphoreType.DMA,
        ],
    )(t2, eflat)


def _edge_body(N, g_ref, ef_ref, dist_ref, dst_ref, we_ref, bm_ref,
               g1_ref, b1_ref, sg_ref, bt_ref, wm_ref, ew_ref, agg_ref):
    R = jnp.dot(ef_ref[0], we_ref[...], preferred_element_type=_F32)
    pre = g_ref[0] + R + bm_ref[0]
    m = _ln(_gelu(pre), g1_ref[0], b1_ref[0])
    d = dist_ref[0, 0, 0]
    sig = sg_ref[0, 0]
    bet = bt_ref[0, 0]
    z = d * d / (2.0 * sig * sig)
    zb = jnp.exp(bet * jnp.log(jnp.maximum(z, 1e-38)))
    ew = jnp.exp(-zb)
    wmv = m * ew[:, None]
    wm_ref[0] = wmv
    ew_ref[0, 0, 0] = ew
    dst = dst_ref[0, 0, 0]
    iota = lax.broadcasted_iota(_I32, (N, dst.shape[0]), 0)
    oh = (iota == dst[None, :]).astype(_F32)
    contrib = jnp.dot(oh, wmv, preferred_element_type=_F32)
    e = pl.program_id(1)

    @pl.when(e == 0)
    def _():
        agg_ref[0] = contrib

    @pl.when(e > 0)
    def _():
        agg_ref[0] += contrib


def _attn_body(F, nodes_ref, part_ref, wq_ref, bq_ref, wk_ref, bk_ref,
               wv_ref, bv_ref, wo_ref, bo_ref, g2_ref, b2_ref, out_ref):
    agg = part_ref[0]
    x = jnp.concatenate([nodes_ref[0], agg], axis=1)
    q = jnp.dot(x, wq_ref[...], preferred_element_type=_F32) + bq_ref[0]
    k = jnp.dot(x, wk_ref[...], preferred_element_type=_F32) + bk_ref[0]
    v = jnp.dot(x, wv_ref[...], preferred_element_type=_F32) + bv_ref[0]
    pd = F // _H
    scale = 1.0 / (pd ** 0.5)
    outs = []
    for h in range(_H):
        sl = slice(h * pd, (h + 1) * pd)
        qh, kh, vh = q[:, sl], k[:, sl], v[:, sl]
        s = lax.dot_general(qh, kh, (((1,), (1,)), ((), ())),
                            preferred_element_type=_F32) * scale
        s = s - jnp.max(s, axis=-1, keepdims=True)
        e = jnp.exp(s)
        w = e / jnp.sum(e, axis=-1, keepdims=True)
        outs.append(jnp.dot(w, vh, preferred_element_type=_F32))
    att = jnp.concatenate(outs, axis=1)
    o = jnp.dot(att, wo_ref[...], preferred_element_type=_F32) + bo_ref[0]
    out_ref[0] = _ln(_gelu(o), g2_ref[0], b2_ref[0])


def kernel(nodes, edge_features, distance, edges, node_mask, W_msg, b_msg,
           ln1_g, ln1_b, Wq, bq, Wk, bk, Wv, bv, Wo, bo, ln2_g, ln2_b,
           sigma, beta):
    B, N, D = nodes.shape
    E = edges.shape[1]
    DE = edge_features.shape[2]
    F = W_msg.shape[1]
    Eb = min(512, E)
    nblk = E // Eb

    T = pl.pallas_call(
        functools.partial(_prep_body, D),
        grid=(B,),
        in_specs=[
            pl.BlockSpec((1, N, D), lambda b: (b, 0, 0)),
            pl.BlockSpec((2 * D + DE, F), lambda b: (0, 0)),
        ],
        out_specs=pl.BlockSpec((1, N, _W128), lambda b: (b, 0, 0)),
        out_shape=jax.ShapeDtypeStruct((B, N, _W128), _F32),
    )(nodes, W_msg)

    t2 = T.reshape(B * N, _W128)
    eflat = edges.reshape(B * E * 2)
    dist4 = distance.reshape(B, nblk, 1, Eb)
    We = W_msg[2 * D:]
    bm = b_msg.reshape(1, F)
    g1, b1 = ln1_g.reshape(1, F), ln1_b.reshape(1, F)
    g2, b2 = ln2_g.reshape(1, F), ln2_b.reshape(1, F)
    sg, bt = sigma.reshape(1, 1), beta.reshape(1, 1)

    G = _sc_gather(B, N, E, F, t2, eflat).reshape(B, E, F)

    dst4 = edges[:, :, 1].reshape(B, nblk, 1, Eb)
    wm, ew4, part4 = pl.pallas_call(
        functools.partial(_edge_body, N),
        grid=(B, nblk),
        in_specs=[
            pl.BlockSpec((1, Eb, F), lambda b, e: (b, e, 0)),
            pl.BlockSpec((1, Eb, DE), lambda b, e: (b, e, 0)),
            pl.BlockSpec((1, 1, 1, Eb), lambda b, e: (b, e, 0, 0)),
            pl.BlockSpec((1, 1, 1, Eb), lambda b, e: (b, e, 0, 0)),
            pl.BlockSpec((DE, F), lambda b, e: (0, 0)),
            pl.BlockSpec((1, F), lambda b, e: (0, 0)),
            pl.BlockSpec((1, F), lambda b, e: (0, 0)),
            pl.BlockSpec((1, F), lambda b, e: (0, 0)),
            pl.BlockSpec((1, 1), lambda b, e: (0, 0)),
            pl.BlockSpec((1, 1), lambda b, e: (0, 0)),
        ],
        out_specs=[
            pl.BlockSpec((1, Eb, F), lambda b, e: (b, e, 0)),
            pl.BlockSpec((1, 1, 1, Eb), lambda b, e: (b, e, 0, 0)),
            pl.BlockSpec((1, N, F), lambda b, e: (b, 0, 0)),
        ],
        out_shape=[
            jax.ShapeDtypeStruct((B, E, F), _F32),
            jax.ShapeDtypeStruct((B, nblk, 1, Eb), _F32),
            jax.ShapeDtypeStruct((B, N, F), _F32),
        ],
    )(G, edge_features, dist4, dst4, We, bm, g1, b1, sg, bt)

    updated = pl.pallas_call(
        functools.partial(_attn_body, F),
        grid=(B,),
        in_specs=[
            pl.BlockSpec((1, N, D), lambda b: (b, 0, 0)),
            pl.BlockSpec((1, N, F), lambda b: (b, 0, 0)),
            pl.BlockSpec((D + F, F), lambda b: (0, 0)),
            pl.BlockSpec((1, F), lambda b: (0, 0)),
            pl.BlockSpec((D + F, F), lambda b: (0, 0)),
            pl.BlockSpec((1, F), lambda b: (0, 0)),
            pl.BlockSpec((D + F, F), lambda b: (0, 0)),
            pl.BlockSpec((1, F), lambda b: (0, 0)),
            pl.BlockSpec((F, F), lambda b: (0, 0)),
            pl.BlockSpec((1, F), lambda b: (0, 0)),
            pl.BlockSpec((1, F), lambda b: (0, 0)),
            pl.BlockSpec((1, F), lambda b: (0, 0)),
        ],
        out_specs=pl.BlockSpec((1, N, F), lambda b: (b, 0, 0)),
        out_shape=jax.ShapeDtypeStruct((B, N, F), _F32),
    )(nodes, part4, Wq, bq.reshape(1, F), Wk, bk.reshape(1, F),
      Wv, bv.reshape(1, F), Wo, bo.reshape(1, F), g2, b2)

    return (updated, wm, distance, edges, ew4.reshape(B, E))


# speedup vs baseline: 1.0156x; 1.0156x over previous
import functools
import jax
import jax.numpy as jnp
from jax import lax
from jax.experimental import pallas as pl
from jax.experimental.pallas import tpu as pltpu
from jax.experimental.pallas import tpu_sc as plsc

_F32 = jnp.float32
_I32 = jnp.int32
_H = 8
_NC = 2
_NS = 16
_NW = _NC * _NS
_W128 = 128


def _erf(x):
    p = 0.3275911
    a1, a2, a3, a4, a5 = (0.254829592, -0.284496736, 1.421413741,
                          -1.453152027, 1.061405429)
    ax = jnp.abs(x)
    t = 1.0 / (1.0 + p * ax)
    poly = ((((a5 * t + a4) * t + a3) * t + a2) * t + a1) * t
    y = 1.0 - poly * jnp.exp(-ax * ax)
    return jnp.sign(x) * y


def _gelu(x):
    return 0.5 * x * (1.0 + _erf(x * 0.7071067811865475))


def _ln(x, g, b, eps=1e-3):
    mu = jnp.mean(x, axis=-1, keepdims=True)
    var = jnp.mean((x - mu) ** 2, axis=-1, keepdims=True)
    return (x - mu) / jnp.sqrt(var + eps) * g + b


def _prep_body(D, nodes_ref, w_ref, t_ref):
    n = nodes_ref[0]
    t_ref[0, :, :D] = jnp.dot(n, w_ref[0:D], preferred_element_type=_F32)
    t_ref[0, :, D:2 * D] = jnp.dot(n, w_ref[D:2 * D],
                                   preferred_element_type=_F32)


def _sc_gather_body(B, N, E, F, CH, t_hbm, e_hbm, g_hbm,
                    e2v, idxv, rows, gbuf, sem):
    epw = (B * E) // _NW
    nch = epw // CH
    wid = lax.axis_index("s") * _NC + lax.axis_index("c")
    base_e = wid * epw
    nbase = (base_e // E) * N

    def chunk(k, carry):
        eoff = base_e + k * CH
        pltpu.sync_copy(e_hbm.at[pl.ds(eoff * 2, 2 * CH)], e2v)

        def mkidx(i, c):
            v = e2v[pl.ds(i * 16, 16)]
            idxv[i // 8, pl.ds((i % 8) * 16, 16)] = v + nbase
            return c

        lax.fori_loop(0, (2 * CH) // 16, mkidx, 0, unroll=4)
        cps = [pltpu.async_copy(t_hbm.at[idxv.at[j]],
                                rows.at[pl.ds(j * 128, 128)], sem)
               for j in range((2 * CH) // 128)]
        for cp in cps:
            cp.wait()

        def pairadd(e, c):
            for u in range(F // 16):
                sl = pl.ds(u * 16, 16)
                gbuf[e, sl] = rows[2 * e, sl] + rows[2 * e + 1,
                                                     pl.ds(F + u * 16, 16)]
            return c

        lax.fori_loop(0, CH, pairadd, 0, unroll=2)
        pltpu.sync_copy(gbuf, g_hbm.at[pl.ds(eoff, CH)])
        return carry

    lax.fori_loop(0, nch, chunk, 0)


def _sc_gather(B, N, E, F, t2, eflat):
    CH = 256
    kfn = functools.partial(_sc_gather_body, B, N, E, F, CH)
    return pl.kernel(
        kfn,
        mesh=plsc.VectorSubcoreMesh(core_axis_name="c", subcore_axis_name="s"),
        out_type=jax.ShapeDtypeStruct((B * E, F), _F32),
        scratch_types=[
            pltpu.VMEM((2 * CH,), _I32),
            pltpu.VMEM(((2 * CH) // 128, 128), _I32),
            pltpu.VMEM((2 * CH, _W128), _F32),
            pltpu.VMEM((CH, F), _F32),
            pltpu.SemaphoreType.DMA,
        ],
    )(t2, eflat)


def _edge_body(N, g_ref, ef_ref, dist_ref, dst_ref, we_ref, bm_ref,
               g1_ref, b1_ref, sg_ref, bt_ref, wm_ref, ew_ref, agg_ref):
    R = jnp.dot(ef_ref[0], we_ref[...], preferred_element_type=_F32)
    pre = g_ref[0] + R + bm_ref[0]
    m = _ln(_gelu(pre), g1_ref[0], b1_ref[0])
    d = dist_ref[0, 0, 0]
    sig = sg_ref[0, 0]
    bet = bt_ref[0, 0]
    z = d * d / (2.0 * sig * sig)
    zb = jnp.exp(bet * jnp.log(jnp.maximum(z, 1e-38)))
    ew = jnp.exp(-zb)
    wmv = m * ew[:, None]
    wm_ref[0] = wmv
    ew_ref[0, 0, 0] = ew
    dst = dst_ref[0, 0, 0]
    iota = lax.broadcasted_iota(_I32, (N, dst.shape[0]), 0)
    oh = (iota == dst[None, :]).astype(jnp.bfloat16)
    contrib = jnp.dot(oh, wmv.astype(jnp.bfloat16),
                      preferred_element_type=_F32)
    e = pl.program_id(1)

    @pl.when(e == 0)
    def _():
        agg_ref[0] = contrib

    @pl.when(e > 0)
    def _():
        agg_ref[0] += contrib


def _attn_body(F, nodes_ref, part_ref, wq_ref, bq_ref, wk_ref, bk_ref,
               wv_ref, bv_ref, wo_ref, bo_ref, g2_ref, b2_ref, out_ref):
    agg = part_ref[0]
    x = jnp.concatenate([nodes_ref[0], agg], axis=1)
    q = jnp.dot(x, wq_ref[...], preferred_element_type=_F32) + bq_ref[0]
    k = jnp.dot(x, wk_ref[...], preferred_element_type=_F32) + bk_ref[0]
    v = jnp.dot(x, wv_ref[...], preferred_element_type=_F32) + bv_ref[0]
    pd = F // _H
    scale = 1.0 / (pd ** 0.5)
    outs = []
    for h in range(_H):
        sl = slice(h * pd, (h + 1) * pd)
        qh, kh, vh = q[:, sl], k[:, sl], v[:, sl]
        s = lax.dot_general(qh, kh, (((1,), (1,)), ((), ())),
                            preferred_element_type=_F32) * scale
        s = s - jnp.max(s, axis=-1, keepdims=True)
        e = jnp.exp(s)
        w = e / jnp.sum(e, axis=-1, keepdims=True)
        outs.append(jnp.dot(w, vh, preferred_element_type=_F32))
    att = jnp.concatenate(outs, axis=1)
    o = jnp.dot(att, wo_ref[...], preferred_element_type=_F32) + bo_ref[0]
    out_ref[0] = _ln(_gelu(o), g2_ref[0], b2_ref[0])


def kernel(nodes, edge_features, distance, edges, node_mask, W_msg, b_msg,
           ln1_g, ln1_b, Wq, bq, Wk, bk, Wv, bv, Wo, bo, ln2_g, ln2_b,
           sigma, beta):
    B, N, D = nodes.shape
    E = edges.shape[1]
    DE = edge_features.shape[2]
    F = W_msg.shape[1]
    Eb = min(512, E)
    nblk = E // Eb

    T = pl.pallas_call(
        functools.partial(_prep_body, D),
        grid=(B,),
        in_specs=[
            pl.BlockSpec((1, N, D), lambda b: (b, 0, 0)),
            pl.BlockSpec((2 * D + DE, F), lambda b: (0, 0)),
        ],
        out_specs=pl.BlockSpec((1, N, _W128), lambda b: (b, 0, 0)),
        out_shape=jax.ShapeDtypeStruct((B, N, _W128), _F32),
    )(nodes, W_msg)

    t2 = T.reshape(B * N, _W128)
    eflat = edges.reshape(B * E * 2)
    dist4 = distance.reshape(B, nblk, 1, Eb)
    We = W_msg[2 * D:]
    bm = b_msg.reshape(1, F)
    g1, b1 = ln1_g.reshape(1, F), ln1_b.reshape(1, F)
    g2, b2 = ln2_g.reshape(1, F), ln2_b.reshape(1, F)
    sg, bt = sigma.reshape(1, 1), beta.reshape(1, 1)

    G = _sc_gather(B, N, E, F, t2, eflat).reshape(B, E, F)

    dst4 = edges[:, :, 1].reshape(B, nblk, 1, Eb)
    wm, ew4, part4 = pl.pallas_call(
        functools.partial(_edge_body, N),
        grid=(B, nblk),
        in_specs=[
            pl.BlockSpec((1, Eb, F), lambda b, e: (b, e, 0)),
            pl.BlockSpec((1, Eb, DE), lambda b, e: (b, e, 0)),
            pl.BlockSpec((1, 1, 1, Eb), lambda b, e: (b, e, 0, 0)),
            pl.BlockSpec((1, 1, 1, Eb), lambda b, e: (b, e, 0, 0)),
            pl.BlockSpec((DE, F), lambda b, e: (0, 0)),
            pl.BlockSpec((1, F), lambda b, e: (0, 0)),
            pl.BlockSpec((1, F), lambda b, e: (0, 0)),
            pl.BlockSpec((1, F), lambda b, e: (0, 0)),
            pl.BlockSpec((1, 1), lambda b, e: (0, 0)),
            pl.BlockSpec((1, 1), lambda b, e: (0, 0)),
        ],
        out_specs=[
            pl.BlockSpec((1, Eb, F), lambda b, e: (b, e, 0)),
            pl.BlockSpec((1, 1, 1, Eb), lambda b, e: (b, e, 0, 0)),
            pl.BlockSpec((1, N, F), lambda b, e: (b, 0, 0)),
        ],
        out_shape=[
            jax.ShapeDtypeStruct((B, E, F), _F32),
            jax.ShapeDtypeStruct((B, nblk, 1, Eb), _F32),
            jax.ShapeDtypeStruct((B, N, F), _F32),
        ],
    )(G, edge_features, dist4, dst4, We, bm, g1, b1, sg, bt)

    updated = pl.pallas_call(
        functools.partial(_attn_body, F),
        grid=(B,),
        in_specs=[
            pl.BlockSpec((1, N, D), lambda b: (b, 0, 0)),
            pl.BlockSpec((1, N, F), lambda b: (b, 0, 0)),
            pl.BlockSpec((D + F, F), lambda b: (0, 0)),
            pl.BlockSpec((1, F), lambda b: (0, 0)),
            pl.BlockSpec((D + F, F), lambda b: (0, 0)),
            pl.BlockSpec((1, F), lambda b: (0, 0)),
            pl.BlockSpec((D + F, F), lambda b: (0, 0)),
            pl.BlockSpec((1, F), lambda b: (0, 0)),
            pl.BlockSpec((F, F), lambda b: (0, 0)),
            pl.BlockSpec((1, F), lambda b: (0, 0)),
            pl.BlockSpec((1, F), lambda b: (0, 0)),
            pl.BlockSpec((1, F), lambda b: (0, 0)),
        ],
        out_specs=pl.BlockSpec((1, N, F), lambda b: (b, 0, 0)),
        out_shape=jax.ShapeDtypeStruct((B, N, F), _F32),
    )(nodes, part4, Wq, bq.reshape(1, F), Wk, bk.reshape(1, F),
      Wv, bv.reshape(1, F), Wo, bo.reshape(1, F), g2, b2)

    return (updated, wm, distance, edges, ew4.reshape(B, E))
